# baseline (speedup 1.0000x reference)
import functools

import jax
import jax.numpy as jnp
from jax import lax
from jax.experimental import pallas as pl
from jax.experimental.pallas import tpu as pltpu
from jax.experimental.pallas import tpu_sc as plsc

N_DIM = 32
N_RATINGS = 5
N_HID = 15
B = 16384
N_USERS = 1000000
N_ITEMS = 100000

_NC = 2
_NS = 16
_NW = _NC * _NS
_BPW = B // _NW
_GSUB = 256
_CHUNK = 32768
_P = _CHUNK // 4


def _tdot(x, e):
    return lax.dot_general(x, e, (((0,), (0,)), ((), ())),
                           preferred_element_type=jnp.float32)


def _pack_body(embT_ref, intT_ref, w1x_ref, bsel_ref, out_ref):
    xe = embT_ref[...].astype(jnp.bfloat16)
    xi = intT_ref[...].astype(jnp.bfloat16)
    w1x = w1x_ref[...].astype(jnp.bfloat16)
    bs = bsel_ref[...].astype(jnp.bfloat16)
    pieces = []
    for w in range(4):
        sl = slice(w * _P, (w + 1) * _P)
        y = _tdot(xe[:, sl], w1x) + _tdot(xi[:, sl], bs)
        pieces.append(pltpu.bitcast(y.astype(jnp.bfloat16), jnp.float32))
    out_ref[...] = jnp.concatenate(pieces, axis=1)


def _pack(embT, intT, w1x, bsel, n):
    g = (n + _CHUNK - 1) // _CHUNK
    return pl.pallas_call(
        _pack_body,
        out_shape=jax.ShapeDtypeStruct((g * (_P // 2), 128), jnp.float32),
        grid=(g,),
        in_specs=[
            pl.BlockSpec((N_DIM, _CHUNK), lambda i: (0, i)),
            pl.BlockSpec((N_RATINGS, _CHUNK), lambda i: (0, i)),
            pl.BlockSpec((N_DIM, 32), lambda i: (0, 0)),
            pl.BlockSpec((N_RATINGS, 32), lambda i: (0, 0)),
        ],
        out_specs=pl.BlockSpec((_P // 2, 128), lambda i: (i, 0)),
    )(embT, intT, w1x, bsel)


def _sc_gather_body(users_hbm, items_hbm, pu_hbm, pi_hbm,
                    rawu_out, rawi_out,
                    idxu_v, idxi_v, rowu_v, rowi_v, gu_v, gi_v,
                    sem0, sem1):
    wid = lax.axis_index("s") * _NC + lax.axis_index("c")
    base = wid * _BPW
    pltpu.sync_copy(users_hbm.at[pl.ds(base, _BPW)], idxu_v)
    pltpu.sync_copy(items_hbm.at[pl.ds(base, _BPW)], idxi_v)
    for k in range(_BPW // 16):
        sl = pl.ds(16 * k, 16)
        iu = idxu_v[sl]
        ii = idxi_v[sl]
        rowu_v[sl] = ((iu >> 15) << 12) + ((iu & 8191) >> 1)
        rowi_v[sl] = ((ii >> 15) << 12) + ((ii & 8191) >> 1)
    for s in range(_BPW // _GSUB):
        sl = pl.ds(s * _GSUB, _GSUB)
        c0 = pltpu.async_copy(pu_hbm.at[rowu_v.at[sl]], gu_v, sem0)
        c1 = pltpu.async_copy(pi_hbm.at[rowi_v.at[sl]], gi_v, sem1)
        out_sl = pl.ds(base + s * _GSUB, _GSUB)
        c0.wait()
        pltpu.sync_copy(gu_v, rawu_out.at[out_sl, :])
        c1.wait()
        pltpu.sync_copy(gi_v, rawi_out.at[out_sl, :])


def _sc_gather(users, items, pu, pi):
    f = functools.partial(
        pl.kernel,
        out_type=(
            jax.ShapeDtypeStruct((B, 128), jnp.float32),
            jax.ShapeDtypeStruct((B, 128), jnp.float32),
        ),
        mesh=plsc.VectorSubcoreMesh(core_axis_name="c",
                                    subcore_axis_name="s"),
        compiler_params=pltpu.CompilerParams(use_tc_tiling_on_sc=False,
                                             needs_layout_passes=False),
        scratch_types=[
            pltpu.VMEM((_BPW,), jnp.int32),
            pltpu.VMEM((_BPW,), jnp.int32),
            pltpu.VMEM((_BPW,), jnp.int32),
            pltpu.VMEM((_BPW,), jnp.int32),
            pltpu.VMEM((_GSUB, 128), jnp.float32),
            pltpu.VMEM((_GSUB, 128), jnp.float32),
            pltpu.SemaphoreType.DMA,
            pltpu.SemaphoreType.DMA,
        ],
    )(_sc_gather_body)
    return f(users, items, pu, pi)


def _tc_mlp_body(rawu_ref, rawi_ref, u_ref, i_ref,
                 b1_ref, w2_ref, b2_ref, out_ref):
    blk = rawu_ref.shape[0]
    vu = lax.bitcast_convert_type(rawu_ref[...], jnp.int32)
    vi = lax.bitcast_convert_type(rawi_ref[...], jnp.int32)
    col = lax.broadcasted_iota(jnp.int32, (blk, 128), 1)
    u = u_ref[0, :].reshape(blk, 1)
    i = i_ref[0, :].reshape(blk, 1)
    hi_mask = jnp.int32(-65536)
    bu = jnp.where((u & 1) == 1, vu & hi_mask, vu << 16)
    bi = jnp.where((i & 1) == 1, vi & hi_mask, vi << 16)
    xu = lax.bitcast_convert_type(bu, jnp.float32)
    xi = lax.bitcast_convert_type(bi, jnp.float32)
    phu = (u >> 13) & 3
    phi = (i >> 13) & 3
    xu = jnp.where((col >> 5) == phu, xu, 0.0)
    xi = jnp.where((col >> 5) == phi, xi, 0.0)
    r = lax.broadcasted_iota(jnp.int32, (128, N_HID), 1)
    sel_h = (lax.broadcasted_iota(jnp.int32, (128, N_HID), 0) % 32
             == r).astype(jnp.float32)
    j = lax.broadcasted_iota(jnp.int32, (128, N_RATINGS), 1)
    sel_s = (lax.broadcasted_iota(jnp.int32, (128, N_RATINGS), 0) % 32
             == 16 + j).astype(jnp.float32)
    hu = jnp.dot(xu, sel_h, preferred_element_type=jnp.float32)
    hi = jnp.dot(xi, sel_h, preferred_element_type=jnp.float32)
    su = jnp.dot(xu, sel_s, preferred_element_type=jnp.float32)
    si = jnp.dot(xi, sel_s, preferred_element_type=jnp.float32)
    h = jnp.maximum(hu + hi + b1_ref[...], 0.0)
    t = jnp.dot(h, w2_ref[...], preferred_element_type=jnp.float32)
    out_ref[...] = t + b2_ref[...] + su + si


def _tc_mlp(rawu, rawi, u2, i2, b1r, w2, b2r):
    blk = 4096
    grid = B // blk
    return pl.pallas_call(
        _tc_mlp_body,
        out_shape=jax.ShapeDtypeStruct((B, N_RATINGS), jnp.float32),
        grid=(grid,),
        in_specs=[
            pl.BlockSpec((blk, 128), lambda i: (i, 0)),
            pl.BlockSpec((blk, 128), lambda i: (i, 0)),
            pl.BlockSpec((1, blk), lambda i: (0, i)),
            pl.BlockSpec((1, blk), lambda i: (0, i)),
            pl.BlockSpec((1, N_HID), lambda i: (0, 0)),
            pl.BlockSpec((N_HID, N_RATINGS), lambda i: (0, 0)),
            pl.BlockSpec((1, N_RATINGS), lambda i: (0, 0)),
        ],
        out_specs=pl.BlockSpec((blk, N_RATINGS), lambda i: (i, 0)),
    )(rawu, rawi, u2, i2, b1r, w2, b2r)


def kernel(users, items, u_emb, i_emb, u_intercept, i_intercept,
           W1, b1, W2, b2):
    zpad = jnp.zeros((N_DIM, 32 - N_HID), jnp.float32)
    w1xu = jnp.concatenate([W1[:N_DIM], zpad], axis=1)
    w1xi = jnp.concatenate([W1[N_DIM:], zpad], axis=1)
    eye5 = jnp.eye(N_RATINGS, dtype=jnp.float32)
    bsel = jnp.concatenate(
        [jnp.zeros((N_RATINGS, 16), jnp.float32), eye5,
         jnp.zeros((N_RATINGS, 32 - 16 - N_RATINGS), jnp.float32)],
        axis=1)
    pu = _pack(u_emb.T, u_intercept.T, w1xu, bsel, N_USERS)
    pi = _pack(i_emb.T, i_intercept.T, w1xi, bsel, N_ITEMS)
    rawu, rawi = _sc_gather(users, items, pu, pi)
    return _tc_mlp(rawu, rawi, users.reshape(1, B), items.reshape(1, B),
                   b1.reshape(1, -1), W2, b2.reshape(1, -1))


# speedup vs baseline: 1.0014x; 1.0014x over previous
import functools

import jax
import jax.numpy as jnp
from jax import lax
from jax.experimental import pallas as pl
from jax.experimental.pallas import tpu as pltpu
from jax.experimental.pallas import tpu_sc as plsc

N_DIM = 32
N_RATINGS = 5
N_HID = 15
B = 16384
N_USERS = 1000000
N_ITEMS = 100000

_NC = 2
_NS = 16
_NW = _NC * _NS
_BPW = B // _NW
_GSUB = 256
_CHUNK = 16384
_P = _CHUNK // 4


def _tdot(x, e):
    return lax.dot_general(x, e, (((0,), (0,)), ((), ())),
                           preferred_element_type=jnp.float32)


def _pack_body(embT_ref, intT_ref, w1x_ref, bsel_ref, out_ref):
    xe = embT_ref[...].astype(jnp.bfloat16)
    xi = intT_ref[...].astype(jnp.bfloat16)
    w1x = w1x_ref[...].astype(jnp.bfloat16)
    bs = bsel_ref[...].astype(jnp.bfloat16)
    pieces = []
    for w in range(4):
        sl = slice(w * _P, (w + 1) * _P)
        y = _tdot(xe[:, sl], w1x) + _tdot(xi[:, sl], bs)
        pieces.append(pltpu.bitcast(y.astype(jnp.bfloat16), jnp.float32))
    out_ref[...] = jnp.concatenate(pieces, axis=1)


def _pack(embT, intT, w1x, bsel, n):
    g = (n + _CHUNK - 1) // _CHUNK
    return pl.pallas_call(
        _pack_body,
        out_shape=jax.ShapeDtypeStruct((g * (_P // 2), 128), jnp.float32),
        grid=(g,),
        in_specs=[
            pl.BlockSpec((N_DIM, _CHUNK), lambda i: (0, i)),
            pl.BlockSpec((N_RATINGS, _CHUNK), lambda i: (0, i)),
            pl.BlockSpec((N_DIM, 32), lambda i: (0, 0)),
            pl.BlockSpec((N_RATINGS, 32), lambda i: (0, 0)),
        ],
        out_specs=pl.BlockSpec((_P // 2, 128), lambda i: (i, 0)),
    )(embT, intT, w1x, bsel)


def _sc_gather_body(users_hbm, items_hbm, pu_hbm, pi_hbm,
                    rawu_out, rawi_out,
                    idxu_v, idxi_v, rowu_v, rowi_v, gu_v, gi_v,
                    sem0, sem1):
    wid = lax.axis_index("s") * _NC + lax.axis_index("c")
    base = wid * _BPW
    pltpu.sync_copy(users_hbm.at[pl.ds(base, _BPW)], idxu_v)
    pltpu.sync_copy(items_hbm.at[pl.ds(base, _BPW)], idxi_v)
    for k in range(_BPW // 16):
        sl = pl.ds(16 * k, 16)
        iu = idxu_v[sl]
        ii = idxi_v[sl]
        rowu_v[sl] = ((iu >> 14) << 11) + ((iu & 4095) >> 1)
        rowi_v[sl] = ((ii >> 14) << 11) + ((ii & 4095) >> 1)
    for s in range(_BPW // _GSUB):
        sl = pl.ds(s * _GSUB, _GSUB)
        c0 = pltpu.async_copy(pu_hbm.at[rowu_v.at[sl]], gu_v, sem0)
        c1 = pltpu.async_copy(pi_hbm.at[rowi_v.at[sl]], gi_v, sem1)
        out_sl = pl.ds(base + s * _GSUB, _GSUB)
        c0.wait()
        pltpu.sync_copy(gu_v, rawu_out.at[out_sl, :])
        c1.wait()
        pltpu.sync_copy(gi_v, rawi_out.at[out_sl, :])


def _sc_gather(users, items, pu, pi):
    f = functools.partial(
        pl.kernel,
        out_type=(
            jax.ShapeDtypeStruct((B, 128), jnp.float32),
            jax.ShapeDtypeStruct((B, 128), jnp.float32),
        ),
        mesh=plsc.VectorSubcoreMesh(core_axis_name="c",
                                    subcore_axis_name="s"),
        compiler_params=pltpu.CompilerParams(use_tc_tiling_on_sc=False,
                                             needs_layout_passes=False),
        scratch_types=[
            pltpu.VMEM((_BPW,), jnp.int32),
            pltpu.VMEM((_BPW,), jnp.int32),
            pltpu.VMEM((_BPW,), jnp.int32),
            pltpu.VMEM((_BPW,), jnp.int32),
            pltpu.VMEM((_GSUB, 128), jnp.float32),
            pltpu.VMEM((_GSUB, 128), jnp.float32),
            pltpu.SemaphoreType.DMA,
            pltpu.SemaphoreType.DMA,
        ],
    )(_sc_gather_body)
    return f(users, items, pu, pi)


def _tc_mlp_body(rawu_ref, rawi_ref, u_ref, i_ref,
                 b1_ref, w2_ref, b2_ref, out_ref):
    blk = rawu_ref.shape[0]
    vu = lax.bitcast_convert_type(rawu_ref[...], jnp.int32)
    vi = lax.bitcast_convert_type(rawi_ref[...], jnp.int32)
    col = lax.broadcasted_iota(jnp.int32, (blk, 128), 1)
    u = u_ref[0, :].reshape(blk, 1)
    i = i_ref[0, :].reshape(blk, 1)
    hi_mask = jnp.int32(-65536)
    bu = jnp.where((u & 1) == 1, vu & hi_mask, vu << 16)
    bi = jnp.where((i & 1) == 1, vi & hi_mask, vi << 16)
    xu = lax.bitcast_convert_type(bu, jnp.float32)
    xi = lax.bitcast_convert_type(bi, jnp.float32)
    phu = (u >> 12) & 3
    phi = (i >> 12) & 3
    xu = jnp.where((col >> 5) == phu, xu, 0.0)
    xi = jnp.where((col >> 5) == phi, xi, 0.0)
    r = lax.broadcasted_iota(jnp.int32, (128, N_HID), 1)
    sel_h = (lax.broadcasted_iota(jnp.int32, (128, N_HID), 0) % 32
             == r).astype(jnp.float32)
    j = lax.broadcasted_iota(jnp.int32, (128, N_RATINGS), 1)
    sel_s = (lax.broadcasted_iota(jnp.int32, (128, N_RATINGS), 0) % 32
             == 16 + j).astype(jnp.float32)
    hu = jnp.dot(xu, sel_h, preferred_element_type=jnp.float32)
    hi = jnp.dot(xi, sel_h, preferred_element_type=jnp.float32)
    su = jnp.dot(xu, sel_s, preferred_element_type=jnp.float32)
    si = jnp.dot(xi, sel_s, preferred_element_type=jnp.float32)
    h = jnp.maximum(hu + hi + b1_ref[...], 0.0)
    t = jnp.dot(h, w2_ref[...], preferred_element_type=jnp.float32)
    out_ref[...] = t + b2_ref[...] + su + si


def _tc_mlp(rawu, rawi, u2, i2, b1r, w2, b2r):
    blk = 2048
    grid = B // blk
    return pl.pallas_call(
        _tc_mlp_body,
        out_shape=jax.ShapeDtypeStruct((B, N_RATINGS), jnp.float32),
        grid=(grid,),
        in_specs=[
            pl.BlockSpec((blk, 128), lambda i: (i, 0)),
            pl.BlockSpec((blk, 128), lambda i: (i, 0)),
            pl.BlockSpec((1, blk), lambda i: (0, i)),
            pl.BlockSpec((1, blk), lambda i: (0, i)),
            pl.BlockSpec((1, N_HID), lambda i: (0, 0)),
            pl.BlockSpec((N_HID, N_RATINGS), lambda i: (0, 0)),
            pl.BlockSpec((1, N_RATINGS), lambda i: (0, 0)),
        ],
        out_specs=pl.BlockSpec((blk, N_RATINGS), lambda i: (i, 0)),
    )(rawu, rawi, u2, i2, b1r, w2, b2r)


def kernel(users, items, u_emb, i_emb, u_intercept, i_intercept,
           W1, b1, W2, b2):
    zpad = jnp.zeros((N_DIM, 32 - N_HID), jnp.float32)
    w1xu = jnp.concatenate([W1[:N_DIM], zpad], axis=1)
    w1xi = jnp.concatenate([W1[N_DIM:], zpad], axis=1)
    eye5 = jnp.eye(N_RATINGS, dtype=jnp.float32)
    bsel = jnp.concatenate(
        [jnp.zeros((N_RATINGS, 16), jnp.float32), eye5,
         jnp.zeros((N_RATINGS, 32 - 16 - N_RATINGS), jnp.float32)],
        axis=1)
    pu = _pack(u_emb.T, u_intercept.T, w1xu, bsel, N_USERS)
    pi = _pack(i_emb.T, i_intercept.T, w1xi, bsel, N_ITEMS)
    rawu, rawi = _sc_gather(users, items, pu, pi)
    return _tc_mlp(rawu, rawi, users.reshape(1, B), items.reshape(1, B),
                   b1.reshape(1, -1), W2, b2.reshape(1, -1))
